# baseline (speedup 1.0000x reference)
import functools

import jax
import jax.numpy as jnp
from jax import lax
from jax.experimental import pallas as pl
from jax.experimental.pallas import tpu as pltpu
from jax.experimental.pallas import tpu_sc as plsc

N = 10000
T = 4
F_IN = 128
HID = 128
GROWTH = 32
E = 320000
THRESH = 0.5
DECAY = 0.25
EPS = 1e-5

NC = 2
NS = 16
NW = NC * NS
CHUNK = 128
CPW = 80
EPW = CPW * CHUNK
EPAD = NW * EPW
NACC = 10240

BN = 400
NB = N // BN

_mesh = plsc.VectorSubcoreMesh(core_axis_name="c", subcore_axis_name="s")


def _deg_body(dstp, ones_hbm, zb, out, dst_v, ones_v, acc):
    c = lax.axis_index("c")
    s = lax.axis_index("s")
    w = s * NC + c
    pltpu.sync_copy(dstp.at[pl.ds(w * CPW, CPW)], dst_v)
    pltpu.sync_copy(ones_hbm, ones_v)
    rpz = NACC // NS
    pltpu.sync_copy(zb, acc.at[pl.ds(s * rpz, rpz)])
    plsc.subcore_barrier()

    def body(i, _):
        pltpu.sync_copy(ones_v, acc.at[dst_v.at[i]], add=True)
        return 0

    lax.fori_loop(0, CPW, body, 0)
    plsc.subcore_barrier()
    pltpu.sync_copy(acc.at[pl.ds(s * rpz, rpz)],
                    out.at[c, pl.ds(s * rpz, rpz)])


@functools.partial(
    pl.kernel,
    out_type=jax.ShapeDtypeStruct((NC, NACC, 128), jnp.float32),
    mesh=_mesh,
    scratch_types=[
        pltpu.VMEM((CPW, CHUNK), jnp.int32),
        pltpu.VMEM((CHUNK, 128), jnp.float32),
        pltpu.VMEM_SHARED((NACC, 128), jnp.float32),
    ],
)
def _sc_degree(dstp, ones_hbm, zb, out, *scratch):
    _deg_body(dstp, ones_hbm, zb, out, *scratch)


G = 16
NG = CPW // G

CPW0 = 80
CPW1 = 80
assert CPW0 + CPW1 == 2 * CPW and CPW0 % G == 0 and CPW1 % G == 0


def _prop_body(table, srcp, dstp, zb, out,
               src_v, dst_v, rows_a, rows_b, acc, sem_a, sem_b):
    c = lax.axis_index("c")
    s = lax.axis_index("s")
    rpz = NACC // NS
    pltpu.sync_copy(zb, acc.at[pl.ds(s * rpz, rpz)])
    plsc.subcore_barrier()

    def g_start(i, buf, sem):
        pltpu.async_copy(table.at[src_v.at[i]], buf, sem)

    def g_wait(buf, sem):
        pltpu.make_async_copy(table.at[src_v.at[0]], buf, sem).wait()

    wbase = (1 - c) * (s * CPW0) + c * (NS * CPW0 + s * CPW1)
    ngrp = (1 - c) * (CPW0 // G) + c * (CPW1 // G)

    def group(g, _):
        base = wbase + g * G
        pltpu.sync_copy(srcp.at[pl.ds(base, G)], src_v)
        pltpu.sync_copy(dstp.at[pl.ds(base, G)], dst_v)
        g_start(0, rows_a, sem_a)

        def body(j, _):
            i0 = 2 * j
            g_start(i0 + 1, rows_b, sem_b)
            g_wait(rows_a, sem_a)
            pltpu.sync_copy(rows_a, acc.at[dst_v.at[i0]], add=True)
            g_start(lax.rem(i0 + 2, G), rows_a, sem_a)
            g_wait(rows_b, sem_b)
            pltpu.sync_copy(rows_b, acc.at[dst_v.at[i0 + 1]], add=True)
            return 0

        lax.fori_loop(0, G // 2, body, 0)
        g_wait(rows_a, sem_a)
        return 0

    lax.fori_loop(0, ngrp, group, 0)
    plsc.subcore_barrier()
    pltpu.sync_copy(acc.at[pl.ds(s * rpz, rpz)],
                    out.at[c, pl.ds(s * rpz, rpz)])


@functools.partial(
    pl.kernel,
    out_type=jax.ShapeDtypeStruct((NC, NACC, 128), jnp.float32),
    mesh=_mesh,
    scratch_types=[
        pltpu.VMEM((G, CHUNK), jnp.int32),
        pltpu.VMEM((G, CHUNK), jnp.int32),
        pltpu.VMEM((CHUNK, 128), jnp.float32),
        pltpu.VMEM((CHUNK, 128), jnp.float32),
        pltpu.VMEM_SHARED((NACC, 128), jnp.float32),
        pltpu.SemaphoreType.DMA,
        pltpu.SemaphoreType.DMA,
    ],
)
def _sc_prop(table, srcp, dstp, zb, out, *scratch):
    _prop_body(table, srcp, dstp, zb, out, *scratch)


def _stats_body(x_ref, o_ref):
    xb = x_ref[...]
    s1 = jnp.sum(xb, axis=(0, 1))
    s2 = jnp.sum(xb * xb, axis=(0, 1))

    @pl.when(pl.program_id(0) == 0)
    def _():
        o_ref[...] = jnp.zeros_like(o_ref)

    o_ref[...] += jnp.stack([s1, s2])


def _stats(x):
    return pl.pallas_call(
        _stats_body,
        grid=(NB,),
        in_specs=[pl.BlockSpec((T, BN, 128), lambda i: (0, i, 0))],
        out_specs=pl.BlockSpec((2, 128), lambda i: (0, 0)),
        out_shape=jax.ShapeDtypeStruct((2, 128), jnp.float32),
    )(x)


def _dinv_body(parts_ref, o_ref):
    deg = parts_ref[0] + parts_ref[1]
    o_ref[...] = jnp.where(deg > 0.0,
                           lax.rsqrt(jnp.maximum(deg, 1e-12)), 0.0)


def _dinv(parts):
    return pl.pallas_call(
        _dinv_body,
        grid=(1,),
        in_specs=[pl.BlockSpec((NC, N, 128), lambda i: (0, 0, 0))],
        out_specs=pl.BlockSpec((N, 128), lambda i: (0, 0)),
        out_shape=jax.ShapeDtypeStruct((N, 128), jnp.float32),
    )(parts)


def _norm_consts(stats_ref, g_ref, be_ref):
    cnt = 1.0 / (T * N)
    mean = stats_ref[0, :] * cnt
    var = stats_ref[1, :] * cnt - mean * mean
    scale = g_ref[0, :] * lax.rsqrt(var + EPS)
    return scale, be_ref[0, :] - mean * scale


def _lif1_body(x_ref, stats_ref, g_ref, be_ref, db_ref, s1_ref, g0_ref):
    scale, off = _norm_consts(stats_ref, g_ref, be_ref)
    db = db_ref[...]
    mem = jnp.zeros((BN, 128), jnp.float32)
    for t in range(T):
        mem = mem * DECAY + (x_ref[t] * scale + off)
        sp = (mem > THRESH).astype(jnp.float32)
        mem = mem * (1.0 - sp)
        s1_ref[t] = sp
        g0_ref[t] = sp * db


def _lif1(x, stats, g1, be1, db):
    return pl.pallas_call(
        _lif1_body,
        grid=(NB,),
        in_specs=[
            pl.BlockSpec((T, BN, 128), lambda i: (0, i, 0)),
            pl.BlockSpec((2, 128), lambda i: (0, 0)),
            pl.BlockSpec((1, 128), lambda i: (0, 0)),
            pl.BlockSpec((1, 128), lambda i: (0, 0)),
            pl.BlockSpec((BN, 128), lambda i: (i, 0)),
        ],
        out_specs=[
            pl.BlockSpec((T, BN, 128), lambda i: (0, i, 0)),
            pl.BlockSpec((T, BN, 128), lambda i: (0, i, 0)),
        ],
        out_shape=[
            jax.ShapeDtypeStruct((T, N, 128), jnp.float32),
            jax.ShapeDtypeStruct((T, N, 128), jnp.float32),
        ],
    )(x, stats, g1, be1, db)


def _conv1_body(s1_ref, p0_ref, p1_ref, p2_ref, p3_ref, db_ref, w_ref, b_ref,
                z_ref, st_ref):
    db = db_ref[...]
    w0 = w_ref[0]
    w1 = w_ref[1]
    b = b_ref[0, :]
    acc = jnp.zeros((2, 128), jnp.float32)
    for t, p_ref in enumerate((p0_ref, p1_ref, p2_ref, p3_ref)):
        ag = db * (p_ref[0] + p_ref[1])
        zt = (jnp.dot(s1_ref[t], w0, preferred_element_type=jnp.float32)
              + jnp.dot(ag, w1, preferred_element_type=jnp.float32) + b)
        z_ref[t] = zt
        acc = acc + jnp.stack([jnp.sum(zt, axis=0), jnp.sum(zt * zt, axis=0)])

    @pl.when(pl.program_id(0) == 0)
    def _():
        st_ref[...] = jnp.zeros_like(st_ref)

    st_ref[...] += acc


def _conv1(s1, props, db, W1, b1):
    blk3 = pl.BlockSpec((T, BN, 128), lambda i: (0, i, 0))
    blkp = pl.BlockSpec((NC, BN, 128), lambda i: (0, i, 0))
    return pl.pallas_call(
        _conv1_body,
        grid=(NB,),
        in_specs=[blk3, blkp, blkp, blkp, blkp,
                  pl.BlockSpec((BN, 128), lambda i: (i, 0)),
                  pl.BlockSpec((2, 128, 128), lambda i: (0, 0, 0)),
                  pl.BlockSpec((1, 128), lambda i: (0, 0))],
        out_specs=[blk3, pl.BlockSpec((2, 128), lambda i: (0, 0))],
        out_shape=[jax.ShapeDtypeStruct((T, N, 128), jnp.float32),
                   jax.ShapeDtypeStruct((2, 128), jnp.float32)],
    )(s1, *props, db, W1, b1)


def _lif2_body(z_ref, stats_ref, g_ref, be_ref, db_ref, w_ref,
               v0_ref, v1_ref, v2_ref, c1_ref):
    scale, off = _norm_consts(stats_ref, g_ref, be_ref)
    d32 = db_ref[:, 0:32]
    mem = jnp.zeros((BN, 128), jnp.float32)
    for t in range(T):
        mem = mem * DECAY + (z_ref[t] * scale + off)
        sp = (mem > THRESH).astype(jnp.float32)
        mem = mem * (1.0 - sp)
        cs = pl.ds(t * 32, 32)
        v0_ref[:, cs] = jnp.dot(sp, w_ref[0], preferred_element_type=jnp.float32)
        v1_ref[:, cs] = d32 * jnp.dot(sp, w_ref[1], preferred_element_type=jnp.float32)
        v2_ref[:, cs] = d32 * jnp.dot(sp, w_ref[2], preferred_element_type=jnp.float32)
        c1_ref[:, cs] = d32 * jnp.dot(sp, w_ref[3], preferred_element_type=jnp.float32)


def _lif2(z1, stats2, g2, be2, db, W2):
    blk = pl.BlockSpec((BN, 128), lambda i: (i, 0))
    return pl.pallas_call(
        _lif2_body,
        grid=(NB,),
        in_specs=[
            pl.BlockSpec((T, BN, 128), lambda i: (0, i, 0)),
            pl.BlockSpec((2, 128), lambda i: (0, 0)),
            pl.BlockSpec((1, 128), lambda i: (0, 0)),
            pl.BlockSpec((1, 128), lambda i: (0, 0)),
            blk,
            pl.BlockSpec((4, 128, 32), lambda i: (0, 0, 0)),
        ],
        out_specs=[blk, blk, blk, blk],
        out_shape=[jax.ShapeDtypeStruct((N, 128), jnp.float32)] * 4,
    )(z1, stats2, g2, be2, db, W2)


def _comb_body(v_ref, q_ref, db_ref, c_ref):
    db = db_ref[...]
    c_ref[...] = v_ref[...] + db * db * (q_ref[0] + q_ref[1])


def _combine(v, q, db):
    blk = pl.BlockSpec((BN, 128), lambda i: (i, 0))
    return pl.pallas_call(
        _comb_body,
        grid=(NB,),
        in_specs=[blk, pl.BlockSpec((NC, BN, 128), lambda i: (0, i, 0)), blk],
        out_specs=blk,
        out_shape=jax.ShapeDtypeStruct((N, 128), jnp.float32),
    )(v, q, db)


def _final_body(x_ref, v0_ref, q3_ref, db_ref, b_ref, o_ref):
    d32 = db_ref[:, 0:32]
    b = b_ref[0, :]
    q = q3_ref[0] + q3_ref[1]
    for t in range(T):
        cs = pl.ds(t * 32, 32)
        o_ref[t, :, 0:128] = x_ref[t]
        o_ref[t, :, 128:160] = v0_ref[:, cs] + d32 * q[:, t * 32:(t + 1) * 32] + b


def _final(x, v0, q3, db, b2):
    return pl.pallas_call(
        _final_body,
        grid=(NB,),
        in_specs=[
            pl.BlockSpec((T, BN, 128), lambda i: (0, i, 0)),
            pl.BlockSpec((BN, 128), lambda i: (i, 0)),
            pl.BlockSpec((NC, BN, 128), lambda i: (0, i, 0)),
            pl.BlockSpec((BN, 128), lambda i: (i, 0)),
            pl.BlockSpec((1, 32), lambda i: (0, 0)),
        ],
        out_specs=pl.BlockSpec((T, BN, 160), lambda i: (0, i, 0)),
        out_shape=jax.ShapeDtypeStruct((T, N, 160), jnp.float32),
    )(x, v0, q3, db, b2)


def kernel(x, edge_index, g1, be1, W1, b1, g2, be2, W2, b2):
    src = edge_index[0]
    dst = edge_index[1]
    pad = EPAD - E
    srcp = jnp.concatenate([src, jnp.zeros((pad,), jnp.int32)]
                           ).reshape(NW * CPW, CHUNK)
    junk = N + (jnp.arange(pad, dtype=jnp.int32) % (NACC - N))
    dstp = jnp.concatenate([dst, junk]).reshape(NW * CPW, CHUNK)
    zb = jnp.zeros((NACC // NS, 128), jnp.float32)
    ones_c = jnp.ones((CHUNK, 128), jnp.float32)

    g1r = g1.reshape(1, 128)
    be1r = be1.reshape(1, 128)
    g2r = g2.reshape(1, 128)
    be2r = be2.reshape(1, 128)
    b1r = b1.reshape(1, 128)
    b2r = b2.reshape(1, 32)

    deg_parts = _sc_degree(dstp, ones_c, zb)
    db = _dinv(deg_parts)
    st1 = _stats(x)
    s1, g0 = _lif1(x, st1, g1r, be1r, db)
    props = [_sc_prop(g0[t], srcp, dstp, zb) for t in range(T)]
    z1, st2 = _conv1(s1, props, db, W1, b1r)
    v0, v1, v2, c1 = _lif2(z1, st2, g2r, be2r, db, W2)
    q1 = _sc_prop(c1, srcp, dstp, zb)
    c2 = _combine(v2, q1, db)
    q2 = _sc_prop(c2, srcp, dstp, zb)
    c3 = _combine(v1, q2, db)
    q3 = _sc_prop(c3, srcp, dstp, zb)
    return _final(x, v0, q3, db, b2r)


# speedup vs baseline: 1.0649x; 1.0649x over previous
import functools

import jax
import jax.numpy as jnp
from jax import lax
from jax.experimental import pallas as pl
from jax.experimental.pallas import tpu as pltpu
from jax.experimental.pallas import tpu_sc as plsc

N = 10000
T = 4
F_IN = 128
HID = 128
GROWTH = 32
E = 320000
THRESH = 0.5
DECAY = 0.25
EPS = 1e-5

NC = 2
NS = 16
NW = NC * NS
CHUNK = 128
CPW = 80
EPW = CPW * CHUNK
EPAD = NW * EPW
NACC = 10240

BN = 400
NB = N // BN

_mesh = plsc.VectorSubcoreMesh(core_axis_name="c", subcore_axis_name="s")


def _deg_body(dstp, ones_hbm, zb, out, dst_v, ones_v, acc):
    c = lax.axis_index("c")
    s = lax.axis_index("s")
    w = s * NC + c
    pltpu.sync_copy(dstp.at[pl.ds(w * CPW, CPW)], dst_v)
    pltpu.sync_copy(ones_hbm, ones_v)
    rpz = NACC // NS
    pltpu.sync_copy(zb, acc.at[pl.ds(s * rpz, rpz)])
    plsc.subcore_barrier()

    def body(i, _):
        pltpu.sync_copy(ones_v, acc.at[dst_v.at[i]], add=True)
        return 0

    lax.fori_loop(0, CPW, body, 0)
    plsc.subcore_barrier()
    pltpu.sync_copy(acc.at[pl.ds(s * rpz, rpz)],
                    out.at[c, pl.ds(s * rpz, rpz)])


@functools.partial(
    pl.kernel,
    out_type=jax.ShapeDtypeStruct((NC, NACC, 128), jnp.float32),
    mesh=_mesh,
    scratch_types=[
        pltpu.VMEM((CPW, CHUNK), jnp.int32),
        pltpu.VMEM((CHUNK, 128), jnp.float32),
        pltpu.VMEM_SHARED((NACC, 128), jnp.float32),
    ],
)
def _sc_degree(dstp, ones_hbm, zb, out, *scratch):
    _deg_body(dstp, ones_hbm, zb, out, *scratch)


G = 16
NG = CPW // G

CPW0 = 128
CPW1 = 32
assert CPW0 + CPW1 == 2 * CPW and CPW0 % G == 0 and CPW1 % G == 0


def _prop_body(table, srcp, dstp, zb, out,
               src_v, dst_v, rows_a, rows_b, acc, sem_a, sem_b):
    c = lax.axis_index("c")
    s = lax.axis_index("s")
    rpz = NACC // NS
    pltpu.sync_copy(zb, acc.at[pl.ds(s * rpz, rpz)])
    plsc.subcore_barrier()

    def g_start(i, buf, sem):
        pltpu.async_copy(table.at[src_v.at[i]], buf, sem)

    def g_wait(buf, sem):
        pltpu.make_async_copy(table.at[src_v.at[0]], buf, sem).wait()

    wbase = (1 - c) * (s * CPW0) + c * (NS * CPW0 + s * CPW1)
    ngrp = (1 - c) * (CPW0 // G) + c * (CPW1 // G)

    def group(g, _):
        base = wbase + g * G
        pltpu.sync_copy(srcp.at[pl.ds(base, G)], src_v)
        pltpu.sync_copy(dstp.at[pl.ds(base, G)], dst_v)
        g_start(0, rows_a, sem_a)

        def body(j, _):
            i0 = 2 * j
            g_start(i0 + 1, rows_b, sem_b)
            g_wait(rows_a, sem_a)
            pltpu.sync_copy(rows_a, acc.at[dst_v.at[i0]], add=True)
            g_start(lax.rem(i0 + 2, G), rows_a, sem_a)
            g_wait(rows_b, sem_b)
            pltpu.sync_copy(rows_b, acc.at[dst_v.at[i0 + 1]], add=True)
            return 0

        lax.fori_loop(0, G // 2, body, 0)
        g_wait(rows_a, sem_a)
        return 0

    lax.fori_loop(0, ngrp, group, 0)
    plsc.subcore_barrier()
    pltpu.sync_copy(acc.at[pl.ds(s * rpz, rpz)],
                    out.at[c, pl.ds(s * rpz, rpz)])


@functools.partial(
    pl.kernel,
    out_type=jax.ShapeDtypeStruct((NC, NACC, 128), jnp.float32),
    mesh=_mesh,
    scratch_types=[
        pltpu.VMEM((G, CHUNK), jnp.int32),
        pltpu.VMEM((G, CHUNK), jnp.int32),
        pltpu.VMEM((CHUNK, 128), jnp.float32),
        pltpu.VMEM((CHUNK, 128), jnp.float32),
        pltpu.VMEM_SHARED((NACC, 128), jnp.float32),
        pltpu.SemaphoreType.DMA,
        pltpu.SemaphoreType.DMA,
    ],
)
def _sc_prop(table, srcp, dstp, zb, out, *scratch):
    _prop_body(table, srcp, dstp, zb, out, *scratch)


def _stats_body(x_ref, o_ref):
    xb = x_ref[...]
    s1 = jnp.sum(xb, axis=(0, 1))
    s2 = jnp.sum(xb * xb, axis=(0, 1))

    @pl.when(pl.program_id(0) == 0)
    def _():
        o_ref[...] = jnp.zeros_like(o_ref)

    o_ref[...] += jnp.stack([s1, s2])


def _stats(x):
    return pl.pallas_call(
        _stats_body,
        grid=(NB,),
        in_specs=[pl.BlockSpec((T, BN, 128), lambda i: (0, i, 0))],
        out_specs=pl.BlockSpec((2, 128), lambda i: (0, 0)),
        out_shape=jax.ShapeDtypeStruct((2, 128), jnp.float32),
    )(x)


def _dinv_body(parts_ref, o_ref):
    deg = parts_ref[0] + parts_ref[1]
    o_ref[...] = jnp.where(deg > 0.0,
                           lax.rsqrt(jnp.maximum(deg, 1e-12)), 0.0)


def _dinv(parts):
    return pl.pallas_call(
        _dinv_body,
        grid=(1,),
        in_specs=[pl.BlockSpec((NC, N, 128), lambda i: (0, 0, 0))],
        out_specs=pl.BlockSpec((N, 128), lambda i: (0, 0)),
        out_shape=jax.ShapeDtypeStruct((N, 128), jnp.float32),
    )(parts)


def _norm_consts(stats_ref, g_ref, be_ref):
    cnt = 1.0 / (T * N)
    mean = stats_ref[0, :] * cnt
    var = stats_ref[1, :] * cnt - mean * mean
    scale = g_ref[0, :] * lax.rsqrt(var + EPS)
    return scale, be_ref[0, :] - mean * scale


def _lif1_body(x_ref, stats_ref, g_ref, be_ref, db_ref, s1_ref, g0_ref):
    scale, off = _norm_consts(stats_ref, g_ref, be_ref)
    db = db_ref[...]
    mem = jnp.zeros((BN, 128), jnp.float32)
    for t in range(T):
        mem = mem * DECAY + (x_ref[t] * scale + off)
        sp = (mem > THRESH).astype(jnp.float32)
        mem = mem * (1.0 - sp)
        s1_ref[t] = sp
        g0_ref[t] = sp * db


def _lif1(x, stats, g1, be1, db):
    return pl.pallas_call(
        _lif1_body,
        grid=(NB,),
        in_specs=[
            pl.BlockSpec((T, BN, 128), lambda i: (0, i, 0)),
            pl.BlockSpec((2, 128), lambda i: (0, 0)),
            pl.BlockSpec((1, 128), lambda i: (0, 0)),
            pl.BlockSpec((1, 128), lambda i: (0, 0)),
            pl.BlockSpec((BN, 128), lambda i: (i, 0)),
        ],
        out_specs=[
            pl.BlockSpec((T, BN, 128), lambda i: (0, i, 0)),
            pl.BlockSpec((T, BN, 128), lambda i: (0, i, 0)),
        ],
        out_shape=[
            jax.ShapeDtypeStruct((T, N, 128), jnp.float32),
            jax.ShapeDtypeStruct((T, N, 128), jnp.float32),
        ],
    )(x, stats, g1, be1, db)


def _conv1_body(s1_ref, p0_ref, p1_ref, p2_ref, p3_ref, db_ref, w_ref, b_ref,
                z_ref, st_ref):
    db = db_ref[...]
    w0 = w_ref[0]
    w1 = w_ref[1]
    b = b_ref[0, :]
    acc = jnp.zeros((2, 128), jnp.float32)
    for t, p_ref in enumerate((p0_ref, p1_ref, p2_ref, p3_ref)):
        ag = db * (p_ref[0] + p_ref[1])
        zt = (jnp.dot(s1_ref[t], w0, preferred_element_type=jnp.float32)
              + jnp.dot(ag, w1, preferred_element_type=jnp.float32) + b)
        z_ref[t] = zt
        acc = acc + jnp.stack([jnp.sum(zt, axis=0), jnp.sum(zt * zt, axis=0)])

    @pl.when(pl.program_id(0) == 0)
    def _():
        st_ref[...] = jnp.zeros_like(st_ref)

    st_ref[...] += acc


def _conv1(s1, props, db, W1, b1):
    blk3 = pl.BlockSpec((T, BN, 128), lambda i: (0, i, 0))
    blkp = pl.BlockSpec((NC, BN, 128), lambda i: (0, i, 0))
    return pl.pallas_call(
        _conv1_body,
        grid=(NB,),
        in_specs=[blk3, blkp, blkp, blkp, blkp,
                  pl.BlockSpec((BN, 128), lambda i: (i, 0)),
                  pl.BlockSpec((2, 128, 128), lambda i: (0, 0, 0)),
                  pl.BlockSpec((1, 128), lambda i: (0, 0))],
        out_specs=[blk3, pl.BlockSpec((2, 128), lambda i: (0, 0))],
        out_shape=[jax.ShapeDtypeStruct((T, N, 128), jnp.float32),
                   jax.ShapeDtypeStruct((2, 128), jnp.float32)],
    )(s1, *props, db, W1, b1)


def _lif2_body(z_ref, stats_ref, g_ref, be_ref, db_ref, w_ref,
               v0_ref, v1_ref, v2_ref, c1_ref):
    scale, off = _norm_consts(stats_ref, g_ref, be_ref)
    d32 = db_ref[:, 0:32]
    mem = jnp.zeros((BN, 128), jnp.float32)
    for t in range(T):
        mem = mem * DECAY + (z_ref[t] * scale + off)
        sp = (mem > THRESH).astype(jnp.float32)
        mem = mem * (1.0 - sp)
        cs = pl.ds(t * 32, 32)
        v0_ref[:, cs] = jnp.dot(sp, w_ref[0], preferred_element_type=jnp.float32)
        v1_ref[:, cs] = d32 * jnp.dot(sp, w_ref[1], preferred_element_type=jnp.float32)
        v2_ref[:, cs] = d32 * jnp.dot(sp, w_ref[2], preferred_element_type=jnp.float32)
        c1_ref[:, cs] = d32 * jnp.dot(sp, w_ref[3], preferred_element_type=jnp.float32)


def _lif2(z1, stats2, g2, be2, db, W2):
    blk = pl.BlockSpec((BN, 128), lambda i: (i, 0))
    return pl.pallas_call(
        _lif2_body,
        grid=(NB,),
        in_specs=[
            pl.BlockSpec((T, BN, 128), lambda i: (0, i, 0)),
            pl.BlockSpec((2, 128), lambda i: (0, 0)),
            pl.BlockSpec((1, 128), lambda i: (0, 0)),
            pl.BlockSpec((1, 128), lambda i: (0, 0)),
            blk,
            pl.BlockSpec((4, 128, 32), lambda i: (0, 0, 0)),
        ],
        out_specs=[blk, blk, blk, blk],
        out_shape=[jax.ShapeDtypeStruct((N, 128), jnp.float32)] * 4,
    )(z1, stats2, g2, be2, db, W2)


def _comb_body(v_ref, q_ref, db_ref, c_ref):
    db = db_ref[...]
    c_ref[...] = v_ref[...] + db * db * (q_ref[0] + q_ref[1])


def _combine(v, q, db):
    blk = pl.BlockSpec((BN, 128), lambda i: (i, 0))
    return pl.pallas_call(
        _comb_body,
        grid=(NB,),
        in_specs=[blk, pl.BlockSpec((NC, BN, 128), lambda i: (0, i, 0)), blk],
        out_specs=blk,
        out_shape=jax.ShapeDtypeStruct((N, 128), jnp.float32),
    )(v, q, db)


def _final_body(x_ref, v0_ref, q3_ref, db_ref, b_ref, o_ref):
    d32 = db_ref[:, 0:32]
    b = b_ref[0, :]
    q = q3_ref[0] + q3_ref[1]
    for t in range(T):
        cs = pl.ds(t * 32, 32)
        o_ref[t, :, 0:128] = x_ref[t]
        o_ref[t, :, 128:160] = v0_ref[:, cs] + d32 * q[:, t * 32:(t + 1) * 32] + b


def _final(x, v0, q3, db, b2):
    return pl.pallas_call(
        _final_body,
        grid=(NB,),
        in_specs=[
            pl.BlockSpec((T, BN, 128), lambda i: (0, i, 0)),
            pl.BlockSpec((BN, 128), lambda i: (i, 0)),
            pl.BlockSpec((NC, BN, 128), lambda i: (0, i, 0)),
            pl.BlockSpec((BN, 128), lambda i: (i, 0)),
            pl.BlockSpec((1, 32), lambda i: (0, 0)),
        ],
        out_specs=pl.BlockSpec((T, BN, 160), lambda i: (0, i, 0)),
        out_shape=jax.ShapeDtypeStruct((T, N, 160), jnp.float32),
    )(x, v0, q3, db, b2)


def kernel(x, edge_index, g1, be1, W1, b1, g2, be2, W2, b2):
    src = edge_index[0]
    dst = edge_index[1]
    pad = EPAD - E
    srcp = jnp.concatenate([src, jnp.zeros((pad,), jnp.int32)]
                           ).reshape(NW * CPW, CHUNK)
    junk = N + (jnp.arange(pad, dtype=jnp.int32) % (NACC - N))
    dstp = jnp.concatenate([dst, junk]).reshape(NW * CPW, CHUNK)
    zb = jnp.zeros((NACC // NS, 128), jnp.float32)
    ones_c = jnp.ones((CHUNK, 128), jnp.float32)

    g1r = g1.reshape(1, 128)
    be1r = be1.reshape(1, 128)
    g2r = g2.reshape(1, 128)
    be2r = be2.reshape(1, 128)
    b1r = b1.reshape(1, 128)
    b2r = b2.reshape(1, 32)

    deg_parts = _sc_degree(dstp, ones_c, zb)
    db = _dinv(deg_parts)
    st1 = _stats(x)
    s1, g0 = _lif1(x, st1, g1r, be1r, db)
    props = [_sc_prop(g0[t], srcp, dstp, zb) for t in range(T)]
    z1, st2 = _conv1(s1, props, db, W1, b1r)
    v0, v1, v2, c1 = _lif2(z1, st2, g2r, be2r, db, W2)
    q1 = _sc_prop(c1, srcp, dstp, zb)
    c2 = _combine(v2, q1, db)
    q2 = _sc_prop(c2, srcp, dstp, zb)
    c3 = _combine(v1, q2, db)
    q3 = _sc_prop(c3, srcp, dstp, zb)
    return _final(x, v0, q3, db, b2r)
